# baseline (speedup 1.0000x reference)
import functools

import jax
import jax.numpy as jnp
from jax import lax
from jax.experimental import pallas as pl
from jax.experimental.pallas import tpu as pltpu
from jax.experimental.pallas import tpu_sc as plsc

T = 16384
D = 2048
NE = 64
K = 8
NW = 32
TPW = T // NW
L = 16
G = TPW // L


_SLABS_PER_BLOCK = 4


def _logits_body(x_ref, w_ref, o_ref):
    for s in range(_SLABS_PER_BLOCK):
        o_ref[s] = lax.dot_general(
            w_ref[...], x_ref[pl.ds(s * TPW, TPW), :],
            dimension_numbers=(((1,), (1,)), ((), ())),
            preferred_element_type=jnp.float32,
        )


_compute_logits = pl.pallas_call(
    _logits_body,
    grid=(NW // _SLABS_PER_BLOCK,),
    in_specs=[
        pl.BlockSpec((_SLABS_PER_BLOCK * TPW, D), lambda i: (i, 0)),
        pl.BlockSpec((NE, D), lambda i: (0, 0)),
    ],
    out_specs=pl.BlockSpec((_SLABS_PER_BLOCK, NE, TPW), lambda i: (i, 0, 0)),
    out_shape=jax.ShapeDtypeStruct((NW, NE, TPW), jnp.float32),
)

_sc_mesh = plsc.VectorSubcoreMesh(core_axis_name="c", subcore_axis_name="s")


@functools.partial(
    pl.kernel,
    mesh=_sc_mesh,
    out_type=[
        jax.ShapeDtypeStruct((T * K,), jnp.int32),
        jax.ShapeDtypeStruct((T * K,), jnp.float32),
    ],
    scratch_types=[
        pltpu.VMEM((NE, TPW), jnp.float32),
        pltpu.VMEM((K * TPW,), jnp.int32),
        pltpu.VMEM((K * TPW,), jnp.float32),
        pltpu.VMEM((TPW * K,), jnp.int32),
        pltpu.VMEM((TPW * K,), jnp.float32),
    ],
    compiler_params=pltpu.CompilerParams(needs_layout_passes=False),
)
def _sc_topk(logits_hbm, idx_hbm, w_hbm, slab, stg_i, stg_w, idx_v, w_v):
    wid = lax.axis_index("s") * 2 + lax.axis_index("c")
    pltpu.sync_copy(logits_hbm.at[wid], slab)

    def group(g, carry):
        base = g * L
        tops = [jnp.full((L,), -jnp.inf, jnp.float32) for _ in range(K)]
        tids = [jnp.zeros((L,), jnp.int32) for _ in range(K)]
        for e in range(NE):
            v = slab[e, pl.ds(base, L)]
            vid = jnp.full((L,), e, jnp.int32)
            for i in range(K):
                m = v > tops[i]
                tv, ti = tops[i], tids[i]
                tops[i] = jnp.where(m, v, tv)
                tids[i] = jnp.where(m, vid, ti)
                v = jnp.where(m, tv, v)
                vid = jnp.where(m, ti, vid)
        mx = tops[0]
        es = [jnp.exp(t - mx) for t in tops]
        s = es[0]
        for i in range(1, K):
            s = s + es[i]
        inv = 1.0 / s
        for i in range(K):
            stg_i[pl.ds(i * TPW + base, L)] = tids[i]
            stg_w[pl.ds(i * TPW + base, L)] = es[i] * inv
        lane = lax.broadcasted_iota(jnp.int32, (L,), 0)
        for j in range(K):
            p = j * L + lane
            src = (p & (K - 1)) * TPW + base + (p >> 3)
            dst = base * K + j * L
            idx_v[pl.ds(dst, L)] = plsc.load_gather(stg_i, [src])
            w_v[pl.ds(dst, L)] = plsc.load_gather(stg_w, [src])
        return carry

    lax.fori_loop(0, G, group, 0)
    el0 = wid * (TPW * K)
    pltpu.sync_copy(idx_v, idx_hbm.at[pl.ds(el0, TPW * K)])
    pltpu.sync_copy(w_v, w_hbm.at[pl.ds(el0, TPW * K)])


def kernel(x, top_k, W):
    del top_k
    logits = _compute_logits(x, W)
    idx, w = _sc_topk(logits)
    return idx.reshape(T, K), w.reshape(T, K)


# speedup vs baseline: 1.0086x; 1.0086x over previous
import functools

import jax
import jax.numpy as jnp
from jax import lax
from jax.experimental import pallas as pl
from jax.experimental.pallas import tpu as pltpu
from jax.experimental.pallas import tpu_sc as plsc

T = 16384
D = 2048
NE = 64
K = 8
NW = 32
TPW = T // NW
L = 16
G = TPW // L


_SLABS_PER_BLOCK = 2


def _logits_body(x0_ref, x1_ref, w_ref, o_ref):
    for s in range(_SLABS_PER_BLOCK):
        acc = lax.dot_general(
            w_ref[:, : D // 2], x0_ref[pl.ds(s * TPW, TPW), :],
            dimension_numbers=(((1,), (1,)), ((), ())),
            preferred_element_type=jnp.float32,
        )
        acc += lax.dot_general(
            w_ref[:, D // 2 :], x1_ref[pl.ds(s * TPW, TPW), :],
            dimension_numbers=(((1,), (1,)), ((), ())),
            preferred_element_type=jnp.float32,
        )
        o_ref[s] = acc


_compute_logits_call = pl.pallas_call(
    _logits_body,
    grid=(NW // _SLABS_PER_BLOCK,),
    in_specs=[
        pl.BlockSpec((_SLABS_PER_BLOCK * TPW, D // 2), lambda i: (i, 0)),
        pl.BlockSpec((_SLABS_PER_BLOCK * TPW, D // 2), lambda i: (i, 1)),
        pl.BlockSpec((NE, D), lambda i: (0, 0)),
    ],
    out_specs=pl.BlockSpec((_SLABS_PER_BLOCK, NE, TPW), lambda i: (i, 0, 0)),
    out_shape=jax.ShapeDtypeStruct((NW, NE, TPW), jnp.float32),
)


def _compute_logits(x, W):
    return _compute_logits_call(x, x, W)

_sc_mesh = plsc.VectorSubcoreMesh(core_axis_name="c", subcore_axis_name="s")


@functools.partial(
    pl.kernel,
    mesh=_sc_mesh,
    out_type=[
        jax.ShapeDtypeStruct((T * K,), jnp.int32),
        jax.ShapeDtypeStruct((T * K,), jnp.float32),
    ],
    scratch_types=[
        pltpu.VMEM((NE, TPW), jnp.float32),
        pltpu.VMEM((K * TPW,), jnp.int32),
        pltpu.VMEM((K * TPW,), jnp.float32),
        pltpu.VMEM((TPW * K,), jnp.int32),
        pltpu.VMEM((TPW * K,), jnp.float32),
    ],
    compiler_params=pltpu.CompilerParams(needs_layout_passes=False),
)
def _sc_topk(logits_hbm, idx_hbm, w_hbm, slab, stg_i, stg_w, idx_v, w_v):
    wid = lax.axis_index("s") * 2 + lax.axis_index("c")
    pltpu.sync_copy(logits_hbm.at[wid], slab)

    def group(g, carry):
        base = g * L
        tops = [jnp.full((L,), -jnp.inf, jnp.float32) for _ in range(K)]
        tids = [jnp.zeros((L,), jnp.int32) for _ in range(K)]
        for e in range(NE):
            v = slab[e, pl.ds(base, L)]
            vid = jnp.full((L,), e, jnp.int32)
            for i in range(K):
                m = v > tops[i]
                tv, ti = tops[i], tids[i]
                tops[i] = jnp.where(m, v, tv)
                tids[i] = jnp.where(m, vid, ti)
                v = jnp.where(m, tv, v)
                vid = jnp.where(m, ti, vid)
        mx = tops[0]
        es = [jnp.exp(t - mx) for t in tops]
        s = es[0]
        for i in range(1, K):
            s = s + es[i]
        inv = 1.0 / s
        for i in range(K):
            stg_i[pl.ds(i * TPW + base, L)] = tids[i]
            stg_w[pl.ds(i * TPW + base, L)] = es[i] * inv
        lane = lax.broadcasted_iota(jnp.int32, (L,), 0)
        for j in range(K):
            p = j * L + lane
            src = (p & (K - 1)) * TPW + base + (p >> 3)
            dst = base * K + j * L
            idx_v[pl.ds(dst, L)] = plsc.load_gather(stg_i, [src])
            w_v[pl.ds(dst, L)] = plsc.load_gather(stg_w, [src])
        return carry

    lax.fori_loop(0, G, group, 0)
    el0 = wid * (TPW * K)
    pltpu.sync_copy(idx_v, idx_hbm.at[pl.ds(el0, TPW * K)])
    pltpu.sync_copy(w_v, w_hbm.at[pl.ds(el0, TPW * K)])


def kernel(x, top_k, W):
    del top_k
    logits = _compute_logits(x, W)
    idx, w = _sc_topk(logits)
    return idx.reshape(T, K), w.reshape(T, K)


# speedup vs baseline: 1.2023x; 1.1920x over previous
import functools

import jax
import jax.numpy as jnp
from jax import lax
from jax.experimental import pallas as pl
from jax.experimental.pallas import tpu as pltpu
from jax.experimental.pallas import tpu_sc as plsc

T = 16384
D = 2048
NE = 64
K = 8
NW = 32
L = 16
BT = 1024
NCHUNKS = 2

_sc_mesh = plsc.VectorSubcoreMesh(core_axis_name="c", subcore_axis_name="s")


def _make_tc_chunk(c, nchunks):
    tc_tokens = T // nchunks
    sw = tc_tokens // NW
    spb = BT // sw
    blocks = tc_tokens // BT

    def body(x_ref, w_ref, o_ref):
        for s in range(spb):
            o_ref[s] = lax.dot_general(
                w_ref[...], x_ref[pl.ds(s * sw, sw), :],
                dimension_numbers=(((1,), (1,)), ((), ())),
                preferred_element_type=jnp.float32,
            )

    return pl.pallas_call(
        body,
        grid=(blocks,),
        in_specs=[
            pl.BlockSpec((BT, D), lambda i, _c=c, _b=blocks: (i + _c * _b, 0)),
            pl.BlockSpec((NE, D), lambda i: (0, 0)),
        ],
        out_specs=pl.BlockSpec((spb, NE, sw), lambda i: (i, 0, 0)),
        out_shape=jax.ShapeDtypeStruct((NW, NE, sw), jnp.float32),
    )


def _make_sc_chunk(nchunks):
    sw = T // nchunks // NW
    groups = sw // L

    @functools.partial(
        pl.kernel,
        mesh=_sc_mesh,
        out_type=[
            jax.ShapeDtypeStruct((NW * sw * K,), jnp.int32),
            jax.ShapeDtypeStruct((NW * sw * K,), jnp.float32),
        ],
        scratch_types=[
            pltpu.VMEM((NE, sw), jnp.float32),
            pltpu.VMEM((K * sw,), jnp.int32),
            pltpu.VMEM((K * sw,), jnp.float32),
            pltpu.VMEM((sw * K,), jnp.int32),
            pltpu.VMEM((sw * K,), jnp.float32),
        ],
        compiler_params=pltpu.CompilerParams(needs_layout_passes=False),
    )
    def sc_topk(logits_hbm, idx_hbm, w_hbm, slab, stg_i, stg_w, idx_v, w_v):
        wid = lax.axis_index("s") * 2 + lax.axis_index("c")
        pltpu.sync_copy(logits_hbm.at[wid], slab)

        def group(g, carry):
            base = g * L
            tops = [jnp.full((L,), -jnp.inf, jnp.float32) for _ in range(K)]
            tids = [jnp.zeros((L,), jnp.int32) for _ in range(K)]
            for e in range(NE):
                v = slab[e, pl.ds(base, L)]
                vid = jnp.full((L,), e, jnp.int32)
                for i in range(K):
                    m = v > tops[i]
                    tv, ti = tops[i], tids[i]
                    tops[i] = jnp.where(m, v, tv)
                    tids[i] = jnp.where(m, vid, ti)
                    v = jnp.where(m, tv, v)
                    vid = jnp.where(m, ti, vid)
            mx = tops[0]
            es = [jnp.exp(t - mx) for t in tops]
            s = es[0]
            for i in range(1, K):
                s = s + es[i]
            inv = 1.0 / s
            for i in range(K):
                stg_i[pl.ds(i * sw + base, L)] = tids[i]
                stg_w[pl.ds(i * sw + base, L)] = es[i] * inv
            lane = lax.broadcasted_iota(jnp.int32, (L,), 0)
            for j in range(K):
                p = j * L + lane
                src = (p & (K - 1)) * sw + base + (p >> 3)
                dst = base * K + j * L
                idx_v[pl.ds(dst, L)] = plsc.load_gather(stg_i, [src])
                w_v[pl.ds(dst, L)] = plsc.load_gather(stg_w, [src])
            return carry

        lax.fori_loop(0, groups, group, 0)
        el0 = wid * (sw * K)
        pltpu.sync_copy(idx_v, idx_hbm.at[pl.ds(el0, sw * K)])
        pltpu.sync_copy(w_v, w_hbm.at[pl.ds(el0, sw * K)])

    return sc_topk


_tc_chunks = [_make_tc_chunk(c, NCHUNKS) for c in range(NCHUNKS)]
_sc_chunk = _make_sc_chunk(NCHUNKS)


def kernel(x, top_k, W):
    del top_k
    outs = []
    for c in range(NCHUNKS):
        logits = _tc_chunks[c](x, W)
        outs.append(_sc_chunk(logits))
    idx = jnp.concatenate([o[0] for o in outs]).reshape(T, K)
    w = jnp.concatenate([o[1] for o in outs]).reshape(T, K)
    return idx, w


# speedup vs baseline: 1.2110x; 1.0073x over previous
import functools

import jax
import jax.numpy as jnp
from jax import lax
from jax.experimental import pallas as pl
from jax.experimental.pallas import tpu as pltpu
from jax.experimental.pallas import tpu_sc as plsc

T = 16384
D = 2048
NE = 64
K = 8
NW = 32
L = 16
BT = 1024
NCHUNKS = 4

_sc_mesh = plsc.VectorSubcoreMesh(core_axis_name="c", subcore_axis_name="s")


def _make_tc_chunk(c, nchunks):
    tc_tokens = T // nchunks
    sw = tc_tokens // NW
    spb = BT // sw
    blocks = tc_tokens // BT

    def body(x_ref, w_ref, o_ref):
        for s in range(spb):
            o_ref[s] = lax.dot_general(
                w_ref[...], x_ref[pl.ds(s * sw, sw), :],
                dimension_numbers=(((1,), (1,)), ((), ())),
                preferred_element_type=jnp.float32,
            )

    return pl.pallas_call(
        body,
        grid=(blocks,),
        in_specs=[
            pl.BlockSpec((BT, D), lambda i, _c=c, _b=blocks: (i + _c * _b, 0)),
            pl.BlockSpec((NE, D), lambda i: (0, 0)),
        ],
        out_specs=pl.BlockSpec((spb, NE, sw), lambda i: (i, 0, 0)),
        out_shape=jax.ShapeDtypeStruct((NW, NE, sw), jnp.float32),
    )


def _make_sc_chunk(nchunks):
    sw = T // nchunks // NW
    groups = sw // L

    @functools.partial(
        pl.kernel,
        mesh=_sc_mesh,
        out_type=[
            jax.ShapeDtypeStruct((NW * sw * K,), jnp.int32),
            jax.ShapeDtypeStruct((NW * sw * K,), jnp.float32),
        ],
        scratch_types=[
            pltpu.VMEM((NE, sw), jnp.float32),
            pltpu.VMEM((K * sw,), jnp.int32),
            pltpu.VMEM((K * sw,), jnp.float32),
            pltpu.VMEM((sw * K,), jnp.int32),
            pltpu.VMEM((sw * K,), jnp.float32),
        ],
        compiler_params=pltpu.CompilerParams(needs_layout_passes=False),
    )
    def sc_topk(logits_hbm, idx_hbm, w_hbm, slab, stg_i, stg_w, idx_v, w_v):
        wid = lax.axis_index("s") * 2 + lax.axis_index("c")
        pltpu.sync_copy(logits_hbm.at[wid], slab)

        def group(g, carry):
            base = g * L
            tops = [jnp.full((L,), -jnp.inf, jnp.float32) for _ in range(K)]
            tids = [jnp.zeros((L,), jnp.int32) for _ in range(K)]
            for e in range(NE):
                v = slab[e, pl.ds(base, L)]
                vid = jnp.full((L,), e, jnp.int32)
                for i in range(K):
                    m = v > tops[i]
                    tv, ti = tops[i], tids[i]
                    tops[i] = jnp.where(m, v, tv)
                    tids[i] = jnp.where(m, vid, ti)
                    v = jnp.where(m, tv, v)
                    vid = jnp.where(m, ti, vid)
            mx = tops[0]
            es = [jnp.exp(t - mx) for t in tops]
            s = es[0]
            for i in range(1, K):
                s = s + es[i]
            inv = 1.0 / s
            for i in range(K):
                stg_i[pl.ds(i * sw + base, L)] = tids[i]
                stg_w[pl.ds(i * sw + base, L)] = es[i] * inv
            lane = lax.broadcasted_iota(jnp.int32, (L,), 0)
            for j in range(K):
                p = j * L + lane
                src = (p & (K - 1)) * sw + base + (p >> 3)
                dst = base * K + j * L
                idx_v[pl.ds(dst, L)] = plsc.load_gather(stg_i, [src])
                w_v[pl.ds(dst, L)] = plsc.load_gather(stg_w, [src])
            return carry

        lax.fori_loop(0, groups, group, 0)
        el0 = wid * (sw * K)
        pltpu.sync_copy(idx_v, idx_hbm.at[pl.ds(el0, sw * K)])
        pltpu.sync_copy(w_v, w_hbm.at[pl.ds(el0, sw * K)])

    return sc_topk


_tc_chunks = [_make_tc_chunk(c, NCHUNKS) for c in range(NCHUNKS)]
_sc_chunk = _make_sc_chunk(NCHUNKS)


def kernel(x, top_k, W):
    del top_k
    outs = []
    for c in range(NCHUNKS):
        logits = _tc_chunks[c](x, W)
        outs.append(_sc_chunk(logits))
    idx = jnp.concatenate([o[0] for o in outs]).reshape(T, K)
    w = jnp.concatenate([o[1] for o in outs]).reshape(T, K)
    return idx, w
